# baseline (speedup 1.0000x reference)
import functools

import jax
import jax.numpy as jnp
from jax import lax
from jax.experimental import pallas as pl
from jax.experimental.pallas import tpu as pltpu
from jax.experimental.pallas import tpu_sc as plsc

_BLK = 512


def _mlp_layer_body(z_ref, wa_ref, ba_ref, wb_ref, bb_ref, out_ref):
    z = z_ref[...]
    t = jnp.maximum(
        jnp.dot(z, wa_ref[...], preferred_element_type=jnp.float32) + ba_ref[...], 0.0)
    o = jnp.maximum(
        jnp.dot(t, wb_ref[...], preferred_element_type=jnp.float32) + bb_ref[...], 0.0)
    out_ref[...] = o


def _mlp_layer(z, wa, ba, wb, bb, interpret=False):
    npad, din = z.shape
    dout = wb.shape[1]
    nb = npad // _BLK
    return pl.pallas_call(
        _mlp_layer_body,
        grid=(nb,),
        in_specs=[
            pl.BlockSpec((_BLK, din), lambda i: (i, 0)),
            pl.BlockSpec((din, 512), lambda i: (0, 0)),
            pl.BlockSpec((1, 512), lambda i: (0, 0)),
            pl.BlockSpec((512, dout), lambda i: (0, 0)),
            pl.BlockSpec((1, dout), lambda i: (0, 0)),
        ],
        out_specs=pl.BlockSpec((_BLK, dout), lambda i: (i, 0)),
        out_shape=jax.ShapeDtypeStruct((npad, dout), jnp.float32),
        interpret=interpret,
    )(z, wa, ba, wb, bb)


def _xw_body(x_ref, w_ref, out_ref):
    out_ref[...] = jnp.dot(x_ref[...], w_ref[...],
                           preferred_element_type=jnp.float32)


def _xw(x, w, interpret=False):
    npad, din = x.shape
    nb = npad // _BLK
    return pl.pallas_call(
        _xw_body,
        grid=(nb,),
        in_specs=[
            pl.BlockSpec((_BLK, din), lambda i: (i, 0)),
            pl.BlockSpec((din, 512), lambda i: (0, 0)),
        ],
        out_specs=pl.BlockSpec((_BLK, 512), lambda i: (i, 0)),
        out_shape=jax.ShapeDtypeStruct((npad, 512), jnp.float32),
        interpret=interpret,
    )(x, w)


def _l1_mlp_body(z_ref, ba_ref, wb_ref, bb_ref, out_ref):
    t = jnp.maximum(z_ref[...] + ba_ref[...], 0.0)
    o = jnp.maximum(
        jnp.dot(t, wb_ref[...], preferred_element_type=jnp.float32) + bb_ref[...], 0.0)
    out_ref[...] = o


def _l1_mlp(z, ba, wb, bb, interpret=False):
    npad = z.shape[0]
    nb = npad // _BLK
    return pl.pallas_call(
        _l1_mlp_body,
        grid=(nb,),
        in_specs=[
            pl.BlockSpec((_BLK, 512), lambda i: (i, 0)),
            pl.BlockSpec((1, 512), lambda i: (0, 0)),
            pl.BlockSpec((512, 512), lambda i: (0, 0)),
            pl.BlockSpec((1, 512), lambda i: (0, 0)),
        ],
        out_specs=pl.BlockSpec((_BLK, 512), lambda i: (i, 0)),
        out_shape=jax.ShapeDtypeStruct((npad, 512), jnp.float32),
        interpret=interpret,
    )(z, ba, wb, bb)


def _l3_pool_body(nblocks, g, z_ref, batch_ref, wa_ref, ba_ref, wb_ref, bb_ref,
                  wl1_ref, bl1_ref, wl2_ref, bl2_ref, out_ref, sums_ref, counts_ref):
    i = pl.program_id(0)

    @pl.when(i == 0)
    def _init():
        sums_ref[...] = jnp.zeros_like(sums_ref)
        counts_ref[...] = jnp.zeros_like(counts_ref)

    z = z_ref[...]
    t = jnp.maximum(
        jnp.dot(z, wa_ref[...], preferred_element_type=jnp.float32) + ba_ref[...], 0.0)
    h3 = jnp.maximum(
        jnp.dot(t, wb_ref[...], preferred_element_type=jnp.float32) + bb_ref[...], 0.0)
    bvec = batch_ref[0]
    gids = lax.broadcasted_iota(jnp.int32, (g, 1), 0)
    onehot = (bvec == gids).astype(jnp.float32)
    sums_ref[...] += jnp.dot(onehot, h3, preferred_element_type=jnp.float32)
    counts_ref[...] += jnp.sum(onehot, axis=1, keepdims=True)

    @pl.when(i == nblocks - 1)
    def _head():
        pooled = sums_ref[...] / jnp.maximum(counts_ref[...], 1.0)
        u = jnp.maximum(
            jnp.dot(pooled, wl1_ref[...], preferred_element_type=jnp.float32)
            + bl1_ref[...], 0.0)
        out_ref[...] = (
            jnp.dot(u, wl2_ref[...], preferred_element_type=jnp.float32) + bl2_ref[...])


def _l3_pool(z, batch3d, wa, ba, wb, bb, wl1, bl1, wl2p, bl2p, g, interpret=False):
    npad = z.shape[0]
    nb = npad // _BLK
    dh = wl2p.shape[1]
    return pl.pallas_call(
        functools.partial(_l3_pool_body, nb, g),
        grid=(nb,),
        in_specs=[
            pl.BlockSpec((_BLK, 512), lambda i: (i, 0)),
            pl.BlockSpec((1, 1, _BLK), lambda i: (i, 0, 0)),
            pl.BlockSpec((512, 512), lambda i: (0, 0)),
            pl.BlockSpec((1, 512), lambda i: (0, 0)),
            pl.BlockSpec((512, 512), lambda i: (0, 0)),
            pl.BlockSpec((1, 512), lambda i: (0, 0)),
            pl.BlockSpec((512, 256), lambda i: (0, 0)),
            pl.BlockSpec((1, 256), lambda i: (0, 0)),
            pl.BlockSpec((256, dh), lambda i: (0, 0)),
            pl.BlockSpec((1, dh), lambda i: (0, 0)),
        ],
        out_specs=pl.BlockSpec((g, dh), lambda i: (0, 0)),
        out_shape=jax.ShapeDtypeStruct((g, dh), jnp.float32),
        scratch_shapes=[
            pltpu.VMEM((g, 512), jnp.float32),
            pltpu.VMEM((g, 1), jnp.float32),
        ],
        interpret=interpret,
    )(z, batch3d, wa, ba, wb, bb, wl1, bl1, wl2p, bl2p)


_NC, _NS, _L = 2, 16, 16
_E = 800000
_EPT = _E // _NS
_W = 2000
_NWIN = _EPT // _W
_RING = 4096
_C = 64
_NPAD = 50176


def _zagg(h, src, dst):
    npad = h.shape[0]
    return h + jax.ops.segment_sum(h[src], dst, num_segments=npad)


def _make_zagg_sc(d, npad=_NPAD, e=_E, wsz=_W, ring=_RING, c=_C, interpret=False):
    half = npad // 2
    rpt = half // _NS
    ept = e // _NS
    nwin = ept // wsz
    mesh = plsc.VectorSubcoreMesh(
        core_axis_name="c", subcore_axis_name="s",
        num_cores=_NC, num_subcores=_NS)

    def body(h_hbm, src_hbm, dst_hbm, out_hbm,
             dstw, srcw, ring_loc, ring_src, idxs_a, idxs_b,
             rows_a, rows_b, gsem_a, gsem_b, ssem_a, ssem_b):
        cid = lax.axis_index("c")
        sid = lax.axis_index("s")
        iota = lax.broadcasted_iota(jnp.int32, (_L,), 0)
        rmask = ring - 1
        lo = cid * half

        def stage(ring, dst_buf, pos):
            for j in range(c // _L):
                dst_buf[pl.ds(j * _L, _L)] = plsc.load_gather(
                    ring, [(pos + j * _L + iota) & rmask])

        def issue(issued):
            par = (issued // c) & 1

            @pl.when(par == 0)
            def _():
                @pl.when(issued >= 2 * c)
                def _():
                    pltpu.make_async_copy(h_hbm.at[pl.ds(0, c)], rows_a,
                                          ssem_a).wait()
                stage(ring_src, idxs_a, issued)
                pltpu.async_copy(h_hbm.at[idxs_a], rows_a, gsem_a)

            @pl.when(par == 1)
            def _():
                @pl.when(issued >= 2 * c)
                def _():
                    pltpu.make_async_copy(h_hbm.at[pl.ds(0, c)], rows_b,
                                          ssem_b).wait()
                stage(ring_src, idxs_b, issued)
                pltpu.async_copy(h_hbm.at[idxs_b], rows_b, gsem_b)

        def retire(drained):
            par = (drained // c) & 1

            @pl.when(par == 0)
            def _():
                pltpu.make_async_copy(h_hbm.at[pl.ds(0, c)], rows_a, gsem_a).wait()
                for j in range(c // _L):
                    loc16 = plsc.load_gather(
                        ring_loc, [(drained + j * _L + iota) & rmask])
                    pltpu.async_copy(rows_a.at[pl.ds(j * _L, _L)],
                                     out_hbm.at[loc16], ssem_a, add=True)

            @pl.when(par == 1)
            def _():
                pltpu.make_async_copy(h_hbm.at[pl.ds(0, c)], rows_b, gsem_b).wait()
                for j in range(c // _L):
                    loc16 = plsc.load_gather(
                        ring_loc, [(drained + j * _L + iota) & rmask])
                    pltpu.async_copy(rows_b.at[pl.ds(j * _L, _L)],
                                     out_hbm.at[loc16], ssem_b, add=True)

        def drain_step(st):
            cnt, iss, drn = st
            drn = lax.cond(iss - drn >= 2 * c,
                           lambda v: (retire(v), v + c)[1],
                           lambda v: v, drn)
            issue(iss)
            return (cnt, iss + c, drn)

        base = lo + sid * rpt
        pltpu.sync_copy(h_hbm.at[pl.ds(base, rpt)], out_hbm.at[pl.ds(base, rpt)])
        plsc.subcore_barrier()

        def window_body(w, st):
            cnt, iss, drn = st
            ebase = sid * ept + w * wsz
            pltpu.sync_copy(dst_hbm.at[pl.ds(ebase, wsz)], dstw)
            pltpu.sync_copy(src_hbm.at[pl.ds(ebase, wsz)], srcw)

            def scan_body(i, cnt):
                dvec = plsc.load_gather(dstw, [i * _L + iota])
                svec = plsc.load_gather(srcw, [i * _L + iota])
                m = (dvec >= lo) & (dvec < lo + half)
                mi = m.astype(jnp.int32)
                pos = (cnt + plsc.cumsum(mi) - 1) & rmask
                plsc.store_scatter(ring_loc, [pos], dvec, mask=m)
                plsc.store_scatter(ring_src, [pos], svec, mask=m)
                return cnt + jnp.sum(mi)

            cnt = lax.fori_loop(0, wsz // _L, scan_body, cnt)
            return lax.while_loop(lambda s: s[0] - s[1] >= c, drain_step,
                                  (cnt, iss, drn))

        cnt, iss, drn = lax.fori_loop(
            0, nwin, window_body, (jnp.int32(0), jnp.int32(0), jnp.int32(0)))

        padn = (c - (cnt - iss)) & (c - 1)
        for j in range(c // _L):
            pm = (j * _L + iota) < padn
            ppos = (cnt + j * _L + iota) & rmask
            plsc.store_scatter(ring_loc, [ppos], npad - 8 + (iota & 7), mask=pm)
            plsc.store_scatter(ring_src, [ppos], iota & 7, mask=pm)
        cnt = cnt + padn
        _, iss, drn = lax.while_loop(lambda s: s[0] - s[1] >= c, drain_step,
                                     (cnt, iss, drn))
        drn = lax.while_loop(lambda v: v < iss,
                             lambda v: (retire(v), v + c)[1], drn)

        @pl.when(iss >= c)
        def _():
            par = ((iss - c) // c) & 1

            @pl.when(par == 0)
            def _():
                pltpu.make_async_copy(h_hbm.at[pl.ds(0, c)], rows_a, ssem_a).wait()

            @pl.when(par == 1)
            def _():
                pltpu.make_async_copy(h_hbm.at[pl.ds(0, c)], rows_b, ssem_b).wait()

        @pl.when(iss >= 2 * c)
        def _():
            par = ((iss - 2 * c) // c) & 1

            @pl.when(par == 0)
            def _():
                pltpu.make_async_copy(h_hbm.at[pl.ds(0, c)], rows_a, ssem_a).wait()

            @pl.when(par == 1)
            def _():
                pltpu.make_async_copy(h_hbm.at[pl.ds(0, c)], rows_b, ssem_b).wait()

    return pl.kernel(
        body,
        out_type=jax.ShapeDtypeStruct((npad, d), jnp.float32),
        mesh=mesh,
        interpret=interpret,
        compiler_params=pltpu.CompilerParams(needs_layout_passes=False),
        scratch_types=[
            pltpu.VMEM((wsz,), jnp.int32),
            pltpu.VMEM((wsz,), jnp.int32),
            pltpu.VMEM((ring,), jnp.int32),
            pltpu.VMEM((ring,), jnp.int32),
            pltpu.VMEM((c,), jnp.int32),
            pltpu.VMEM((c,), jnp.int32),
            pltpu.VMEM((c, d), jnp.float32),
            pltpu.VMEM((c, d), jnp.float32),
            pltpu.SemaphoreType.DMA,
            pltpu.SemaphoreType.DMA,
            pltpu.SemaphoreType.DMA,
            pltpu.SemaphoreType.DMA,
        ],
    )


_zagg_sc_512 = _make_zagg_sc(512)


def kernel(x, edge_index, batch, w1a, b1a, w1b, b1b, w2a, b2a, w2b, b2b,
           w3a, b3a, w3b, b3b, wl1, bl1, wl2, bl2):
    n = x.shape[0]
    npad = _NPAD
    src = edge_index[0]
    dst = edge_index[1]

    xp = jnp.zeros((npad, 16), jnp.float32).at[:n, :7].set(x)
    w1a_p = jnp.zeros((16, 512), jnp.float32).at[:7, :].set(w1a)

    b1a_r, b1b_r = b1a[None, :], b1b[None, :]
    b2a_r, b2b_r = b2a[None, :], b2b[None, :]
    b3a_r, b3b_r = b3a[None, :], b3b[None, :]
    bl1_r = bl1[None, :]
    dh = 128
    wl2p = jnp.zeros((256, dh), jnp.float32).at[:, :12].set(wl2)
    bl2p = jnp.zeros((1, dh), jnp.float32).at[0, :12].set(bl2)

    batch3d = jnp.full((npad,), 128, jnp.int32).at[:n].set(batch).reshape(
        npad // _BLK, 1, _BLK)

    y = _xw(xp, w1a_p)
    z1 = _zagg(y, src, dst)
    h1 = _l1_mlp(z1, b1a_r, w1b, b1b_r)
    z2 = _zagg(h1, src, dst)
    h2 = _mlp_layer(z2, w2a, b2a_r, w2b, b2b_r)
    z3 = _zagg(h2, src, dst)
    out = _l3_pool(z3, batch3d, w3a, b3a_r, w3b, b3b_r, wl1, bl1_r, wl2p, bl2p, 128)
    return out[:, :12]


# speedup vs baseline: 1.1944x; 1.1944x over previous
import functools

import jax
import jax.numpy as jnp
from jax import lax
from jax.experimental import pallas as pl
from jax.experimental.pallas import tpu as pltpu
from jax.experimental.pallas import tpu_sc as plsc

_BLK = 512


def _mlp_layer_body(z_ref, wa_ref, ba_ref, wb_ref, bb_ref, out_ref):
    z = z_ref[...]
    t = jnp.maximum(
        jnp.dot(z, wa_ref[...], preferred_element_type=jnp.float32) + ba_ref[...], 0.0)
    o = jnp.maximum(
        jnp.dot(t, wb_ref[...], preferred_element_type=jnp.float32) + bb_ref[...], 0.0)
    out_ref[...] = o


def _mlp_layer(z, wa, ba, wb, bb, interpret=False):
    npad, din = z.shape
    dout = wb.shape[1]
    nb = npad // _BLK
    return pl.pallas_call(
        _mlp_layer_body,
        grid=(nb,),
        in_specs=[
            pl.BlockSpec((_BLK, din), lambda i: (i, 0)),
            pl.BlockSpec((din, 512), lambda i: (0, 0)),
            pl.BlockSpec((1, 512), lambda i: (0, 0)),
            pl.BlockSpec((512, dout), lambda i: (0, 0)),
            pl.BlockSpec((1, dout), lambda i: (0, 0)),
        ],
        out_specs=pl.BlockSpec((_BLK, dout), lambda i: (i, 0)),
        out_shape=jax.ShapeDtypeStruct((npad, dout), jnp.float32),
        interpret=interpret,
    )(z, wa, ba, wb, bb)


def _xw_body(x_ref, w_ref, out_ref):
    out_ref[...] = jnp.dot(x_ref[...], w_ref[...],
                           preferred_element_type=jnp.float32)


def _xw(x, w, interpret=False):
    npad, din = x.shape
    nb = npad // _BLK
    return pl.pallas_call(
        _xw_body,
        grid=(nb,),
        in_specs=[
            pl.BlockSpec((_BLK, din), lambda i: (i, 0)),
            pl.BlockSpec((din, 512), lambda i: (0, 0)),
        ],
        out_specs=pl.BlockSpec((_BLK, 512), lambda i: (i, 0)),
        out_shape=jax.ShapeDtypeStruct((npad, 512), jnp.float32),
        interpret=interpret,
    )(x, w)


def _l1_mlp_body(z_ref, ba_ref, wb_ref, bb_ref, out_ref):
    t = jnp.maximum(z_ref[...] + ba_ref[...], 0.0)
    o = jnp.maximum(
        jnp.dot(t, wb_ref[...], preferred_element_type=jnp.float32) + bb_ref[...], 0.0)
    out_ref[...] = o


def _l1_mlp(z, ba, wb, bb, interpret=False):
    npad = z.shape[0]
    nb = npad // _BLK
    return pl.pallas_call(
        _l1_mlp_body,
        grid=(nb,),
        in_specs=[
            pl.BlockSpec((_BLK, 512), lambda i: (i, 0)),
            pl.BlockSpec((1, 512), lambda i: (0, 0)),
            pl.BlockSpec((512, 512), lambda i: (0, 0)),
            pl.BlockSpec((1, 512), lambda i: (0, 0)),
        ],
        out_specs=pl.BlockSpec((_BLK, 512), lambda i: (i, 0)),
        out_shape=jax.ShapeDtypeStruct((npad, 512), jnp.float32),
        interpret=interpret,
    )(z, ba, wb, bb)


def _l3_pool_body(nblocks, g, z_ref, batch_ref, wa_ref, ba_ref, wb_ref, bb_ref,
                  wl1_ref, bl1_ref, wl2_ref, bl2_ref, out_ref, sums_ref, counts_ref):
    i = pl.program_id(0)

    @pl.when(i == 0)
    def _init():
        sums_ref[...] = jnp.zeros_like(sums_ref)
        counts_ref[...] = jnp.zeros_like(counts_ref)

    z = z_ref[...]
    t = jnp.maximum(
        jnp.dot(z, wa_ref[...], preferred_element_type=jnp.float32) + ba_ref[...], 0.0)
    h3 = jnp.maximum(
        jnp.dot(t, wb_ref[...], preferred_element_type=jnp.float32) + bb_ref[...], 0.0)
    bvec = batch_ref[0]
    gids = lax.broadcasted_iota(jnp.int32, (g, 1), 0)
    onehot = (bvec == gids).astype(jnp.float32)
    sums_ref[...] += jnp.dot(onehot, h3, preferred_element_type=jnp.float32)
    counts_ref[...] += jnp.sum(onehot, axis=1, keepdims=True)

    @pl.when(i == nblocks - 1)
    def _head():
        pooled = sums_ref[...] / jnp.maximum(counts_ref[...], 1.0)
        u = jnp.maximum(
            jnp.dot(pooled, wl1_ref[...], preferred_element_type=jnp.float32)
            + bl1_ref[...], 0.0)
        out_ref[...] = (
            jnp.dot(u, wl2_ref[...], preferred_element_type=jnp.float32) + bl2_ref[...])


def _l3_pool(z, batch3d, wa, ba, wb, bb, wl1, bl1, wl2p, bl2p, g, interpret=False):
    npad = z.shape[0]
    nb = npad // _BLK
    dh = wl2p.shape[1]
    return pl.pallas_call(
        functools.partial(_l3_pool_body, nb, g),
        grid=(nb,),
        in_specs=[
            pl.BlockSpec((_BLK, 512), lambda i: (i, 0)),
            pl.BlockSpec((1, 1, _BLK), lambda i: (i, 0, 0)),
            pl.BlockSpec((512, 512), lambda i: (0, 0)),
            pl.BlockSpec((1, 512), lambda i: (0, 0)),
            pl.BlockSpec((512, 512), lambda i: (0, 0)),
            pl.BlockSpec((1, 512), lambda i: (0, 0)),
            pl.BlockSpec((512, 256), lambda i: (0, 0)),
            pl.BlockSpec((1, 256), lambda i: (0, 0)),
            pl.BlockSpec((256, dh), lambda i: (0, 0)),
            pl.BlockSpec((1, dh), lambda i: (0, 0)),
        ],
        out_specs=pl.BlockSpec((g, dh), lambda i: (0, 0)),
        out_shape=jax.ShapeDtypeStruct((g, dh), jnp.float32),
        scratch_shapes=[
            pltpu.VMEM((g, 512), jnp.float32),
            pltpu.VMEM((g, 1), jnp.float32),
        ],
        interpret=interpret,
    )(z, batch3d, wa, ba, wb, bb, wl1, bl1, wl2p, bl2p)


_NC, _NS, _L = 2, 16, 16
_E = 800000
_EPT = _E // _NS
_W = 2000
_NWIN = _EPT // _W
_RING = 4096
_C = 64
_NPAD = 50176


def _zagg(h, src, dst):
    npad = h.shape[0]
    return h + jax.ops.segment_sum(h[src], dst, num_segments=npad)


def _make_zagg_sc(d, npad=_NPAD, e=_E, wsz=_W, ring=_RING, c=_C, interpret=False):
    half = npad // 2
    rpt = half // _NS
    ept = e // _NS
    nwin = ept // wsz
    mesh = plsc.VectorSubcoreMesh(
        core_axis_name="c", subcore_axis_name="s",
        num_cores=_NC, num_subcores=_NS)

    def body(h_hbm, src_hbm, dst_hbm, out_hbm,
             dstw, srcw, ring_loc, ring_src, idxs_a, idxs_b,
             rows_a, rows_b, gsem_a, gsem_b, ssem_a, ssem_b):
        cid = lax.axis_index("c")
        sid = lax.axis_index("s")
        iota = lax.broadcasted_iota(jnp.int32, (_L,), 0)
        rmask = ring - 1
        lo = cid * half

        def stage(ring, dst_buf, pos):
            for j in range(c // _L):
                dst_buf[pl.ds(j * _L, _L)] = plsc.load_gather(
                    ring, [(pos + j * _L + iota) & rmask])

        def issue(issued):
            par = (issued // c) & 1

            @pl.when(par == 0)
            def _():
                @pl.when(issued >= 2 * c)
                def _():
                    pltpu.make_async_copy(h_hbm.at[pl.ds(0, c)], rows_a,
                                          ssem_a).wait()
                stage(ring_src, idxs_a, issued)
                pltpu.async_copy(h_hbm.at[idxs_a], rows_a, gsem_a)

            @pl.when(par == 1)
            def _():
                @pl.when(issued >= 2 * c)
                def _():
                    pltpu.make_async_copy(h_hbm.at[pl.ds(0, c)], rows_b,
                                          ssem_b).wait()
                stage(ring_src, idxs_b, issued)
                pltpu.async_copy(h_hbm.at[idxs_b], rows_b, gsem_b)

        def retire(drained):
            par = (drained // c) & 1

            @pl.when(par == 0)
            def _():
                pltpu.make_async_copy(h_hbm.at[pl.ds(0, c)], rows_a, gsem_a).wait()
                for j in range(c // _L):
                    loc16 = plsc.load_gather(
                        ring_loc, [(drained + j * _L + iota) & rmask])
                    pltpu.async_copy(rows_a.at[pl.ds(j * _L, _L)],
                                     out_hbm.at[loc16], ssem_a, add=True)

            @pl.when(par == 1)
            def _():
                pltpu.make_async_copy(h_hbm.at[pl.ds(0, c)], rows_b, gsem_b).wait()
                for j in range(c // _L):
                    loc16 = plsc.load_gather(
                        ring_loc, [(drained + j * _L + iota) & rmask])
                    pltpu.async_copy(rows_b.at[pl.ds(j * _L, _L)],
                                     out_hbm.at[loc16], ssem_b, add=True)

        def drain_step(st):
            cnt, iss, drn = st
            drn = lax.cond(iss - drn >= 2 * c,
                           lambda v: (retire(v), v + c)[1],
                           lambda v: v, drn)
            issue(iss)
            return (cnt, iss + c, drn)

        base = lo + sid * rpt
        pltpu.sync_copy(h_hbm.at[pl.ds(base, rpt)], out_hbm.at[pl.ds(base, rpt)])
        plsc.subcore_barrier()

        def window_body(w, st):
            cnt, iss, drn = st
            ebase = sid * ept + w * wsz
            pltpu.sync_copy(dst_hbm.at[pl.ds(ebase, wsz)], dstw)
            pltpu.sync_copy(src_hbm.at[pl.ds(ebase, wsz)], srcw)

            def scan_body(i, cnt):
                dvec = plsc.load_gather(dstw, [i * _L + iota])
                svec = plsc.load_gather(srcw, [i * _L + iota])
                m = (dvec >= lo) & (dvec < lo + half)
                mi = m.astype(jnp.int32)
                pos = (cnt + plsc.cumsum(mi) - 1) & rmask
                plsc.store_scatter(ring_loc, [pos], dvec, mask=m)
                plsc.store_scatter(ring_src, [pos], svec, mask=m)
                return cnt + jnp.sum(mi)

            cnt = lax.fori_loop(0, wsz // _L, scan_body, cnt)
            return lax.while_loop(lambda s: s[0] - s[1] >= c, drain_step,
                                  (cnt, iss, drn))

        cnt, iss, drn = lax.fori_loop(
            0, nwin, window_body, (jnp.int32(0), jnp.int32(0), jnp.int32(0)))

        padn = (c - (cnt - iss)) & (c - 1)
        for j in range(c // _L):
            pm = (j * _L + iota) < padn
            ppos = (cnt + j * _L + iota) & rmask
            plsc.store_scatter(ring_loc, [ppos], npad - 8 + (iota & 7), mask=pm)
            plsc.store_scatter(ring_src, [ppos], iota & 7, mask=pm)
        cnt = cnt + padn
        _, iss, drn = lax.while_loop(lambda s: s[0] - s[1] >= c, drain_step,
                                     (cnt, iss, drn))
        drn = lax.while_loop(lambda v: v < iss,
                             lambda v: (retire(v), v + c)[1], drn)

        @pl.when(iss >= c)
        def _():
            par = ((iss - c) // c) & 1

            @pl.when(par == 0)
            def _():
                pltpu.make_async_copy(h_hbm.at[pl.ds(0, c)], rows_a, ssem_a).wait()

            @pl.when(par == 1)
            def _():
                pltpu.make_async_copy(h_hbm.at[pl.ds(0, c)], rows_b, ssem_b).wait()

        @pl.when(iss >= 2 * c)
        def _():
            par = ((iss - 2 * c) // c) & 1

            @pl.when(par == 0)
            def _():
                pltpu.make_async_copy(h_hbm.at[pl.ds(0, c)], rows_a, ssem_a).wait()

            @pl.when(par == 1)
            def _():
                pltpu.make_async_copy(h_hbm.at[pl.ds(0, c)], rows_b, ssem_b).wait()

    return pl.kernel(
        body,
        out_type=jax.ShapeDtypeStruct((npad, d), jnp.float32),
        mesh=mesh,
        interpret=interpret,
        compiler_params=pltpu.CompilerParams(needs_layout_passes=False),
        scratch_types=[
            pltpu.VMEM((wsz,), jnp.int32),
            pltpu.VMEM((wsz,), jnp.int32),
            pltpu.VMEM((ring,), jnp.int32),
            pltpu.VMEM((ring,), jnp.int32),
            pltpu.VMEM((c,), jnp.int32),
            pltpu.VMEM((c,), jnp.int32),
            pltpu.VMEM((c, d), jnp.float32),
            pltpu.VMEM((c, d), jnp.float32),
            pltpu.SemaphoreType.DMA,
            pltpu.SemaphoreType.DMA,
            pltpu.SemaphoreType.DMA,
            pltpu.SemaphoreType.DMA,
        ],
    )


_zagg_sc_512 = _make_zagg_sc(512)


def kernel(x, edge_index, batch, w1a, b1a, w1b, b1b, w2a, b2a, w2b, b2b,
           w3a, b3a, w3b, b3b, wl1, bl1, wl2, bl2):
    n = x.shape[0]
    npad = _NPAD
    src = edge_index[0]
    dst = edge_index[1]

    xp = jnp.zeros((npad, 16), jnp.float32).at[:n, :7].set(x)
    w1a_p = jnp.zeros((16, 512), jnp.float32).at[:7, :].set(w1a)

    b1a_r, b1b_r = b1a[None, :], b1b[None, :]
    b2a_r, b2b_r = b2a[None, :], b2b[None, :]
    b3a_r, b3b_r = b3a[None, :], b3b[None, :]
    bl1_r = bl1[None, :]
    dh = 128
    wl2p = jnp.zeros((256, dh), jnp.float32).at[:, :12].set(wl2)
    bl2p = jnp.zeros((1, dh), jnp.float32).at[0, :12].set(bl2)

    batch3d = jnp.full((npad,), 128, jnp.int32).at[:n].set(batch).reshape(
        npad // _BLK, 1, _BLK)

    z1 = _zagg(xp, src, dst)
    h1 = _mlp_layer(z1, w1a_p, b1a_r, w1b, b1b_r)
    z2 = _zagg(h1, src, dst)
    h2 = _mlp_layer(z2, w2a, b2a_r, w2b, b2b_r)
    z3 = _zagg(h2, src, dst)
    out = _l3_pool(z3, batch3d, w3a, b3a_r, w3b, b3b_r, wl1, bl1_r, wl2p, bl2p, 128)
    return out[:, :12]
